# baseline (speedup 1.0000x reference)
import functools

import jax
import jax.numpy as jnp
from jax.experimental import pallas as pl
from jax.experimental.pallas import tpu as pltpu

_N = 10000
_E = 160000
_IN = 2
_HID = 64
_K = 2
_SEQ = 12
_B = 2
_LAYERS = 2
_NM = 2 * _K + 1


def _gate_body(rg_ref, ug_ref, h_ref, rh_ref, u_ref):
    rh_ref[...] = jax.nn.sigmoid(rg_ref[...]) * h_ref[...]
    u_ref[...] = jax.nn.sigmoid(ug_ref[...])


def _gru_gates(rg, ug, h):
    shp = jax.ShapeDtypeStruct(rg.shape, rg.dtype)
    return pl.pallas_call(
        _gate_body,
        out_shape=(shp, shp),
    )(rg, ug, h)


def _update_body(cg_ref, u_ref, h_ref, out_ref):
    c = jnp.tanh(cg_ref[...])
    u = u_ref[...]
    out_ref[...] = u * h_ref[...] + (1.0 - u) * c


def _gru_update(cg, u, h):
    return pl.pallas_call(
        _update_body,
        out_shape=jax.ShapeDtypeStruct(cg.shape, cg.dtype),
    )(cg, u, h)


def kernel(inputs, init_state, edge_weight, Wg0, bg0, Wc0, bc0, Wg1, bg1, Wc1, bc1, edge_index):
    src = edge_index[0]
    dst = edge_index[1]
    deg_out = jax.ops.segment_sum(edge_weight, src, num_segments=_N) + 1e-8
    deg_in = jax.ops.segment_sum(edge_weight, dst, num_segments=_N) + 1e-8
    w_fw = edge_weight / deg_out[src]
    w_bw = edge_weight / deg_in[dst]

    def spmm_fw(x):
        return jax.ops.segment_sum(x[src] * w_fw[:, None], dst, num_segments=_N)

    def spmm_bw(x):
        return jax.ops.segment_sum(x[dst] * w_bw[:, None], src, num_segments=_N)

    def gconv(x_cat, W, b, out_dim):
        Bsz = x_cat.shape[0]
        C = x_cat.shape[2]
        x0 = jnp.transpose(x_cat, (1, 2, 0)).reshape(_N, C * Bsz)
        xs = [x0]
        for spmm in (spmm_fw, spmm_bw):
            x1 = spmm(x0)
            xs.append(x1)
            xkm2, xkm1 = x0, x1
            for _ in range(2, _K + 1):
                x2 = 2.0 * spmm(xkm1) - xkm2
                xs.append(x2)
                xkm2, xkm1 = xkm1, x2
        xm = jnp.stack(xs, axis=0)
        xm = xm.reshape(_NM, _N, C, Bsz).transpose(3, 1, 2, 0).reshape(Bsz * _N, C * _NM)
        out = xm @ W + b
        return out.reshape(Bsz, _N * out_dim)

    def cell(x_t, h, Wg, bg, Wc, bc, in_dim):
        xr = x_t.reshape(_B, _N, in_dim)
        hr = h.reshape(_B, _N, _HID)
        cat = jnp.concatenate([xr, hr], axis=-1)
        gates = gconv(cat, Wg, bg, 2 * _HID).reshape(_B, _N, 2 * _HID)
        rg, ug = jnp.split(gates, 2, axis=-1)
        rg = rg.reshape(_B, _N * _HID)
        ug = ug.reshape(_B, _N * _HID)
        rh, u = _gru_gates(rg, ug, h)
        cat2 = jnp.concatenate([xr, rh.reshape(_B, _N, _HID)], axis=-1)
        cg = gconv(cat2, Wc, bc, _HID)
        return _gru_update(cg, u, h)

    seq_in = inputs.reshape(_SEQ, _B, _N * _IN)
    layer_params = [(Wg0, bg0, Wc0, bc0, _IN), (Wg1, bg1, Wc1, bc1, _HID)]
    current = seq_in
    context = []
    for li in range(_LAYERS):
        Wg, bg, Wc, bc, ind = layer_params[li]
        h = init_state[li]
        outs = []
        for t in range(_SEQ):
            h = cell(current[t], h, Wg, bg, Wc, bc, ind)
            outs.append(h)
        context.append(h)
        current = jnp.stack(outs, axis=0)
    return jnp.stack(context, axis=0), current


# speedup vs baseline: 2.3186x; 2.3186x over previous
import functools

import jax
import jax.numpy as jnp
from jax import lax
from jax.experimental import pallas as pl
from jax.experimental.pallas import tpu as pltpu
from jax.experimental.pallas import tpu_sc as plsc

_N = 10000
_N2 = 10240
_E = 160000
_IN = 2
_HID = 64
_SEQ = 12
_B = 2
_NM = 5

_W0 = 128
_C0 = _IN + _HID
_W1 = 128

_CH = 80
_EPT = _E // 16
_SCH = 2000
_NSUP = _EPT // _SCH
_NCH = _SCH // _CH
_RPT = _N2 // 16
_ZR = 128
_BLK = 512


def _make_gconv_kernel(W):
    mesh = plsc.VectorSubcoreMesh(core_axis_name="c", subcore_axis_name="s")
    osh = jax.ShapeDtypeStruct((2 * _N2, W), jnp.float32)
    jw = W // 16

    @functools.partial(
        pl.kernel,
        mesh=mesh,
        out_type=(osh, osh, osh, osh),
        scratch_types=[
            pltpu.VMEM((1, _SCH), jnp.int32),
            pltpu.VMEM((1, _SCH), jnp.int32),
            pltpu.VMEM((1, _SCH), jnp.float32),
            pltpu.VMEM((_CH,), jnp.int32),
            pltpu.VMEM((_CH,), jnp.int32),
            pltpu.VMEM((_CH, W), jnp.float32),
            pltpu.VMEM((_ZR, W), jnp.float32),
            pltpu.VMEM_SHARED((_N2, W), jnp.float32),
            pltpu.SemaphoreType.DMA,
        ],
    )
    def kern(x0, srce, dste, wfw, wbw, x1f, tf, x1b, tb,
             gall, sall, wall, gidx, sidx, rows, zbuf, acc, sem):
        c = lax.axis_index("c")
        s = lax.axis_index("s")
        coff = c * _N2

        def zrow(r, carry):
            for j in range(jw):
                zbuf[r, pl.ds(j * 16, 16)] = jnp.zeros((16,), jnp.float32)
            return carry

        lax.fori_loop(0, _ZR, zrow, 0)

        def phase(g_h, s_h, w_h, tab_h, out_h):
            for z in range(_RPT // _ZR):
                pltpu.sync_copy(zbuf, acc.at[pl.ds(s * _RPT + z * _ZR, _ZR)])
            plsc.subcore_barrier()

            def sup(k, kcarry):
                row = s * _NSUP + k
                pltpu.sync_copy(g_h.at[row], gall)
                pltpu.sync_copy(s_h.at[row], sall)
                pltpu.sync_copy(w_h.at[row], wall)

                def chunk(e, carry):
                    b0 = e * _CH
                    for g in range(_CH // 16):
                        gidx[pl.ds(g * 16, 16)] = (
                            gall[0, pl.ds(b0 + g * 16, 16)] + coff
                        )
                        sidx[pl.ds(g * 16, 16)] = (
                            sall[0, pl.ds(b0 + g * 16, 16)]
                        )
                    pltpu.async_copy(tab_h.at[gidx], rows, sem).wait()

                    def scale(g, gcarry):
                        wv = wall[0, pl.ds(b0 + g * 16, 16)]
                        for l in range(16):
                            wi = wv[l]
                            i = g * 16 + l
                            for j in range(jw):
                                rows[i, pl.ds(j * 16, 16)] = (
                                    rows[i, pl.ds(j * 16, 16)] * wi
                                )
                        return gcarry

                    lax.fori_loop(0, _CH // 16, scale, 0)
                    pltpu.sync_copy(rows, acc.at[sidx], add=True)
                    return carry

                lax.fori_loop(0, _NCH, chunk, 0)
                return kcarry

            lax.fori_loop(0, _NSUP, sup, 0)
            plsc.subcore_barrier()
            for z in range(_RPT // _ZR):
                r0 = s * _RPT + z * _ZR
                pltpu.sync_copy(
                    acc.at[pl.ds(r0, _ZR)], out_h.at[pl.ds(coff + r0, _ZR)]
                )
            plsc.subcore_barrier()

        phase(srce, dste, wfw, x0, x1f)
        phase(srce, dste, wfw, x1f, tf)
        phase(dste, srce, wbw, x0, x1b)
        phase(dste, srce, wbw, x1b, tb)

    return kern


_gconv0 = _make_gconv_kernel(_W0)
_gconv1 = _gconv0


def _gates_body(x0r, x1r, x2r, x3r, x4r, hr, vr, br, rhr, ur):
    acc = jnp.dot(x0r[...], vr[0], preferred_element_type=jnp.float32)
    acc += jnp.dot(x1r[...], vr[1], preferred_element_type=jnp.float32)
    acc += jnp.dot(x2r[...], vr[2], preferred_element_type=jnp.float32)
    acc += jnp.dot(x3r[...], vr[3], preferred_element_type=jnp.float32)
    acc += jnp.dot(x4r[...], vr[4], preferred_element_type=jnp.float32)
    g = jax.nn.sigmoid(acc + br[...])
    rhr[...] = g[:, :_HID] * hr[...]
    ur[...] = g[:, _HID:]


def _tc_gates(xs, h, v, b):
    W = xs[0].shape[1]
    grid = (2 * _N2) // _BLK
    xspec = pl.BlockSpec((_BLK, W), lambda i: (i, 0))
    hsh = jax.ShapeDtypeStruct((2 * _N2, _HID), jnp.float32)
    return pl.pallas_call(
        _gates_body,
        grid=(grid,),
        in_specs=[xspec] * 5 + [
            pl.BlockSpec((_BLK, _HID), lambda i: (i, 0)),
            pl.BlockSpec((_NM, W, 2 * _HID), lambda i: (0, 0, 0)),
            pl.BlockSpec((1, 2 * _HID), lambda i: (0, 0)),
        ],
        out_specs=[pl.BlockSpec((_BLK, _HID), lambda i: (i, 0))] * 2,
        out_shape=(hsh, hsh),
    )(*xs, h, v, b)


def _cand_body(x0r, x1r, x2r, x3r, x4r, ur, hr, vr, br, outr):
    acc = jnp.dot(x0r[...], vr[0], preferred_element_type=jnp.float32)
    acc += jnp.dot(x1r[...], vr[1], preferred_element_type=jnp.float32)
    acc += jnp.dot(x2r[...], vr[2], preferred_element_type=jnp.float32)
    acc += jnp.dot(x3r[...], vr[3], preferred_element_type=jnp.float32)
    acc += jnp.dot(x4r[...], vr[4], preferred_element_type=jnp.float32)
    cnd = jnp.tanh(acc + br[...])
    u = ur[...]
    outr[...] = u * hr[...] + (1.0 - u) * cnd


def _tc_cand(xs, u, h, v, b):
    W = xs[0].shape[1]
    grid = (2 * _N2) // _BLK
    xspec = pl.BlockSpec((_BLK, W), lambda i: (i, 0))
    hspec = pl.BlockSpec((_BLK, _HID), lambda i: (i, 0))
    return pl.pallas_call(
        _cand_body,
        grid=(grid,),
        in_specs=[xspec] * 5 + [
            hspec,
            hspec,
            pl.BlockSpec((_NM, W, _HID), lambda i: (0, 0, 0)),
            pl.BlockSpec((1, _HID), lambda i: (0, 0)),
        ],
        out_specs=pl.BlockSpec((_BLK, _HID), lambda i: (i, 0)),
        out_shape=jax.ShapeDtypeStruct((2 * _N2, _HID), jnp.float32),
    )(*xs, u, h, v, b)


def _fold_weights(Wmat, C, WP, out_dim):
    v = Wmat.reshape(C, _NM, out_dim).transpose(1, 0, 2)
    v0 = v[0] - v[2] - v[4]
    vf = jnp.stack([v0, v[1], 2.0 * v[2], v[3], 2.0 * v[4]])
    if WP > C:
        vf = jnp.pad(vf, ((0, 0), (0, WP - C), (0, 0)))
    return vf


def _pad_nodes(x):
    return jnp.pad(x, ((0, 0), (0, _N2 - _N), (0, 0))).reshape(2 * _N2, -1)


def kernel(inputs, init_state, edge_weight, Wg0, bg0, Wc0, bc0,
           Wg1, bg1, Wc1, bc1, edge_index):
    src = edge_index[0]
    dst = edge_index[1]
    deg_out = jax.ops.segment_sum(edge_weight, src, num_segments=_N) + 1e-8
    deg_in = jax.ops.segment_sum(edge_weight, dst, num_segments=_N) + 1e-8
    nrow = _E // _SCH
    w_fw = (edge_weight / deg_out[src]).reshape(nrow, 1, _SCH)
    w_bw = (edge_weight / deg_in[dst]).reshape(nrow, 1, _SCH)
    srcr = src.reshape(nrow, 1, _SCH)
    dstr = dst.reshape(nrow, 1, _SCH)

    vg0 = _fold_weights(Wg0, _C0, _W0, 2 * _HID)
    vc0 = _fold_weights(Wc0, _C0, _W0, _HID)
    vg1 = _fold_weights(Wg1, _W1, _W1, 2 * _HID)
    vc1 = _fold_weights(Wc1, _W1, _W1, _HID)
    bg0r = bg0.reshape(1, 2 * _HID)
    bc0r = bc0.reshape(1, _HID)
    bg1r = bg1.reshape(1, 2 * _HID)
    bc1r = bc1.reshape(1, _HID)

    def cell(xpart, h, gconv, vg, bg, vc, bc, pad):
        if pad:
            zp = jnp.zeros((2 * _N2, pad), jnp.float32)
            x0 = jnp.concatenate([xpart, h, zp], axis=1)
        else:
            x0 = jnp.concatenate([xpart, h], axis=1)
        x1f, tf, x1b, tb = gconv(x0, srcr, dstr, w_fw, w_bw)
        rh, u = _tc_gates([x0, x1f, tf, x1b, tb], h, vg, bg)
        if pad:
            zp = jnp.zeros((2 * _N2, pad), jnp.float32)
            c0 = jnp.concatenate([xpart, rh, zp], axis=1)
        else:
            c0 = jnp.concatenate([xpart, rh], axis=1)
        c1f, ctf, c1b, ctb = gconv(c0, srcr, dstr, w_fw, w_bw)
        return _tc_cand([c0, c1f, ctf, c1b, ctb], u, h, vc, bc)

    seq_in = inputs.reshape(_SEQ, _B, _N, _IN)
    layer_cfg = [
        (_gconv0, vg0, bg0r, vc0, bc0r, _W0 - _C0),
        (_gconv1, vg1, bg1r, vc1, bc1r, 0),
    ]
    context = []
    current = [_pad_nodes(seq_in[t]) for t in range(_SEQ)]
    for li in range(2):
        gconv, vg, bg, vc, bc, pad = layer_cfg[li]
        h = _pad_nodes(init_state[li].reshape(_B, _N, _HID))
        outs = []
        for t in range(_SEQ):
            h = cell(current[t], h, gconv, vg, bg, vc, bc, pad)
            outs.append(h)
        context.append(h)
        current = outs

    def unpad(x):
        return x.reshape(_B, _N2, _HID)[:, :_N].reshape(_B, _N * _HID)

    ctx = jnp.stack([unpad(hh) for hh in context], axis=0)
    cur = jnp.stack([unpad(o) for o in outs], axis=0)
    return ctx, cur
